# baseline (speedup 1.0000x reference)
import functools
import jax
import jax.numpy as jnp
import numpy as np
from jax.experimental import pallas as pl
from jax.experimental.pallas import tpu as pltpu

_RB = 2000
_INTERPRET = False


def _pc(body, grid, in_specs, out_specs, out_shape):
    return pl.pallas_call(
        body, grid=grid, in_specs=in_specs, out_specs=out_specs,
        out_shape=out_shape, interpret=_INTERPRET)


def _row_spec(cols):
    return pl.BlockSpec((_RB, cols), lambda i: (i, 0))


def _full_spec(shape):
    return pl.BlockSpec(shape, lambda i: (0, 0))


def _featurize_body(bf_ref, bx_ref, o_ref):
    pieces = []
    for i in range(3):
        f = bf_ref[:, i:i + 1]
        oh = (f == jax.lax.broadcasted_iota(jnp.int32, (1, 8), 1))
        pieces.append(oh.astype(jnp.float32))
    x = bx_ref[:, 0:1]
    cent = (jax.lax.broadcasted_iota(jnp.int32, (1, 20), 1)
            .astype(jnp.float32) * (2.0 / 19.0))
    e = jnp.exp(-10.0 * (x - cent) ** 2)
    z = jnp.zeros((bf_ref.shape[0], 4), jnp.float32)
    o_ref[...] = jnp.concatenate(pieces + [e, z], axis=1)


def _featurize(bond_feats, bond_float, rows):
    return _pc(
        _featurize_body, (rows // _RB,),
        [_row_spec(3), _row_spec(1)], _row_spec(48),
        jax.ShapeDtypeStruct((rows, 48), jnp.float32),
    )(bond_feats, bond_float)


def _atom_body(af_ref, w_ref, o_ref):
    pieces = []
    for i in range(9):
        f = af_ref[:, i:i + 1]
        oh = (f == jax.lax.broadcasted_iota(jnp.int32, (1, 16), 1))
        pieces.append(oh.astype(jnp.float32))
    x = jnp.concatenate(pieces, axis=1)
    o_ref[...] = jnp.dot(x, w_ref[...], preferred_element_type=jnp.float32)


def _atom_embed(atom_feats, w, rows):
    return _pc(
        _atom_body, (rows // _RB,),
        [_row_spec(9), _full_spec((144, 32))], _row_spec(32),
        jax.ShapeDtypeStruct((rows, 32), jnp.float32),
    )(atom_feats, w)


def _mm_body(x_ref, w_ref, b_ref, o_ref):
    o_ref[...] = (jnp.dot(x_ref[...], w_ref[...],
                          preferred_element_type=jnp.float32) + b_ref[...])


def _mm(x, w, b, rows, k):
    return _pc(
        _mm_body, (rows // _RB,),
        [_row_spec(k), _full_spec((k, 32)), _full_spec((1, 32))],
        _row_spec(32),
        jax.ShapeDtypeStruct((rows, 32), jnp.float32),
    )(x, w, b)


def _ba_embed_body(x_ref, w_ref, b_ref, o_ref):
    x = x_ref[:, 0:1]
    cent = (jax.lax.broadcasted_iota(jnp.int32, (1, 20), 1)
            .astype(jnp.float32) * (2.0 / 19.0))
    e = jnp.exp(-10.0 * (x - cent) ** 2)
    o_ref[...] = (jnp.dot(e, w_ref[...],
                          preferred_element_type=jnp.float32) + b_ref[...])


def _ba_embed(x, w, b, rows):
    return _pc(
        _ba_embed_body, (rows // _RB,),
        [_row_spec(1), _full_spec((20, 32)), _full_spec((1, 32))],
        _row_spec(32),
        jax.ShapeDtypeStruct((rows, 32), jnp.float32),
    )(x, w, b)


def _gin_post_body(n_ref, a0_ref, a1_ref, w1_ref, b1_ref, w2_ref, b2_ref,
                   g_ref, bt_ref, o_ref, *, inv_sqrt_n):
    node = n_ref[...]
    h = node + a0_ref[...] + a1_ref[...]
    a = jnp.maximum(jnp.dot(h, w1_ref[...],
                            preferred_element_type=jnp.float32) + b1_ref[...],
                    0.0)
    z = (jnp.dot(a, w2_ref[...], preferred_element_type=jnp.float32)
         + b2_ref[...])
    m = jnp.mean(z, axis=-1, keepdims=True)
    v = jnp.mean((z - m) ** 2, axis=-1, keepdims=True)
    ln = (z - m) / jnp.sqrt(v + 1e-5) * g_ref[...] + bt_ref[...]
    o_ref[...] = jnp.maximum(ln * inv_sqrt_n, 0.0) + node


def _gin_post(node, a0, a1, p, n_norm, rows):
    body = functools.partial(_gin_post_body,
                             inv_sqrt_n=float(1.0 / np.sqrt(n_norm)))
    return _pc(
        body, (rows // _RB,),
        [_row_spec(32), _row_spec(32), _row_spec(32),
         _full_spec((32, 64)), _full_spec((1, 64)),
         _full_spec((64, 32)), _full_spec((1, 32)),
         _full_spec((1, 32)), _full_spec((1, 32))],
        _row_spec(32),
        jax.ShapeDtypeStruct((rows, 32), jnp.float32),
    )(node, a0, a1,
      p["mlp1"]["w"], p["mlp1"]["b"].reshape(1, 64),
      p["mlp2"]["w"], p["mlp2"]["b"].reshape(1, 32),
      p["gamma"].reshape(1, 32), p["beta"].reshape(1, 32))


def _mean_body(x_ref, o_ref):
    @pl.when(pl.program_id(0) == 0)
    def _():
        o_ref[...] = jnp.zeros_like(o_ref)
    o_ref[...] += jnp.sum(x_ref[...], axis=0, keepdims=True)


def _mean_pool(x, rows):
    s = _pc(
        _mean_body, (rows // _RB,),
        [_row_spec(32)], pl.BlockSpec((1, 32), lambda i: (0, 0)),
        jax.ShapeDtypeStruct((1, 32), jnp.float32),
    )(x)
    return s / float(rows)


def _concat_bond_tables(tables, rbf_lin):
    g = jnp.concatenate([jnp.concatenate(list(tables), axis=0),
                         rbf_lin["w"],
                         jnp.zeros((4, 32), jnp.float32)], axis=0)
    return g, rbf_lin["b"].reshape(1, 32)


def kernel(atom_feats, bond_feats, bond_float_feats, bond_angle_float_feats,
           ab_edge_index, ba_edge_index, params):
    n_atoms = atom_feats.shape[0]
    e_ab = bond_feats.shape[0]
    e_ba = bond_angle_float_feats.shape[0]
    n_bonds = e_ab // 2

    src_ab, dst_ab = ab_edge_index[0], ab_edge_index[1]
    src_ba, dst_ba = ba_edge_index[0], ba_edge_index[1]
    eidx2 = jnp.arange(e_ab, dtype=jnp.int32) // 2

    atom_w = jnp.concatenate(list(params["init_atom_emb"]), axis=0)
    node_repr = _atom_embed(atom_feats, atom_w, n_atoms)

    feats = _featurize(bond_feats, bond_float_feats, e_ab)
    feats_even = feats[::2]

    g0, b0 = _concat_bond_tables(params["init_bond_emb"],
                                 params["init_bond_rbf"])
    edge_full = _mm(feats, g0, b0, e_ab, 48)

    zeros_atom = jnp.zeros((n_atoms, 32), jnp.float32)
    zeros_bond = jnp.zeros((n_bonds, 32), jnp.float32)

    edge_half = None
    for lp in params["layers"]:
        edge_term = edge_full if edge_half is None else edge_half[eidx2]
        msg = jnp.maximum(node_repr[src_ab] + edge_term, 0.0)
        agg = jax.ops.segment_sum(msg, dst_ab, num_segments=n_atoms)
        node_out = _gin_post(node_repr, agg, zeros_atom, lp["ab_gnn"],
                             n_atoms, n_atoms)

        gl, bl = _concat_bond_tables(lp["bond_emb"], lp["bond_rbf"])
        bond_embed = _mm(feats_even, gl, bl, n_bonds, 48)
        ba_emb = _ba_embed(bond_angle_float_feats,
                           lp["ba_rbf"]["w"], lp["ba_rbf"]["b"].reshape(1, 32),
                           e_ba)
        msg_b = jnp.maximum(bond_embed[src_ba] + ba_emb, 0.0)
        agg_b = jax.ops.segment_sum(msg_b, dst_ba, num_segments=n_bonds)
        edge_half = _gin_post(bond_embed, agg_b, zeros_bond, lp["ba_gnn"],
                              n_bonds, n_bonds)

        node_repr = node_out

    edge_repr = jnp.repeat(edge_half, 2, axis=0)
    graph_repr = _mean_pool(node_repr, n_atoms)
    return node_repr, edge_repr, graph_repr


# speedup vs baseline: 16.7592x; 16.7592x over previous
import functools
import jax
import jax.numpy as jnp
import numpy as np
from jax import lax
from jax.experimental import pallas as pl
from jax.experimental.pallas import tpu as pltpu
from jax.experimental.pallas import tpu_sc as plsc

_RB = 2000
_INTERPRET = False


def _pc(body, grid, in_specs, out_specs, out_shape):
    return pl.pallas_call(
        body, grid=grid, in_specs=in_specs, out_specs=out_specs,
        out_shape=out_shape, interpret=_INTERPRET)


def _row_spec(cols):
    return pl.BlockSpec((_RB, cols), lambda i: (i, 0))


def _full_spec(shape):
    return pl.BlockSpec(shape, lambda i: (0, 0))


def _featurize_body(bf_ref, bx_ref, o_ref):
    pieces = []
    for i in range(3):
        f = bf_ref[:, i:i + 1]
        oh = (f == jax.lax.broadcasted_iota(jnp.int32, (1, 8), 1))
        pieces.append(oh.astype(jnp.float32))
    x = bx_ref[:, 0:1]
    cent = (jax.lax.broadcasted_iota(jnp.int32, (1, 20), 1)
            .astype(jnp.float32) * (2.0 / 19.0))
    e = jnp.exp(-10.0 * (x - cent) ** 2)
    z = jnp.zeros((bf_ref.shape[0], 4), jnp.float32)
    o_ref[...] = jnp.concatenate(pieces + [e, z], axis=1)


def _featurize(bond_feats, bond_float, rows):
    return _pc(
        _featurize_body, (rows // _RB,),
        [_row_spec(3), _row_spec(1)], _row_spec(48),
        jax.ShapeDtypeStruct((rows, 48), jnp.float32),
    )(bond_feats, bond_float)


def _atom_body(af_ref, w_ref, o_ref):
    pieces = []
    for i in range(9):
        f = af_ref[:, i:i + 1]
        oh = (f == jax.lax.broadcasted_iota(jnp.int32, (1, 16), 1))
        pieces.append(oh.astype(jnp.float32))
    x = jnp.concatenate(pieces, axis=1)
    o_ref[...] = jnp.dot(x, w_ref[...], preferred_element_type=jnp.float32)


def _atom_embed(atom_feats, w, rows):
    return _pc(
        _atom_body, (rows // _RB,),
        [_row_spec(9), _full_spec((144, 32))], _row_spec(32),
        jax.ShapeDtypeStruct((rows, 32), jnp.float32),
    )(atom_feats, w)


def _mm_body(x_ref, w_ref, b_ref, o_ref):
    o_ref[...] = (jnp.dot(x_ref[...], w_ref[...],
                          preferred_element_type=jnp.float32) + b_ref[...])


def _mm(x, w, b, rows, k):
    return _pc(
        _mm_body, (rows // _RB,),
        [_row_spec(k), _full_spec((k, 32)), _full_spec((1, 32))],
        _row_spec(32),
        jax.ShapeDtypeStruct((rows, 32), jnp.float32),
    )(x, w, b)


def _ba_embed_body(x_ref, w_ref, b_ref, o_ref):
    x = x_ref[:, 0:1]
    cent = (jax.lax.broadcasted_iota(jnp.int32, (1, 20), 1)
            .astype(jnp.float32) * (2.0 / 19.0))
    e = jnp.exp(-10.0 * (x - cent) ** 2)
    o_ref[...] = (jnp.dot(e, w_ref[...],
                          preferred_element_type=jnp.float32) + b_ref[...])


def _ba_embed(x, w, b, rows):
    return _pc(
        _ba_embed_body, (rows // _RB,),
        [_row_spec(1), _full_spec((20, 32)), _full_spec((1, 32))],
        _row_spec(32),
        jax.ShapeDtypeStruct((rows, 32), jnp.float32),
    )(x, w, b)


def _gin_post_body(n_ref, *refs, inv_sqrt_n, n_agg):
    agg_refs = refs[:n_agg]
    (w1_ref, b1_ref, w2_ref, b2_ref, g_ref, bt_ref, o_ref) = refs[n_agg:]
    node = n_ref[...]
    h = node
    for a_ref in agg_refs:
        h = h + a_ref[...]
    a = jnp.maximum(jnp.dot(h, w1_ref[...],
                            preferred_element_type=jnp.float32) + b1_ref[...],
                    0.0)
    z = (jnp.dot(a, w2_ref[...], preferred_element_type=jnp.float32)
         + b2_ref[...])
    m = jnp.mean(z, axis=-1, keepdims=True)
    v = jnp.mean((z - m) ** 2, axis=-1, keepdims=True)
    ln = (z - m) / jnp.sqrt(v + 1e-5) * g_ref[...] + bt_ref[...]
    o_ref[...] = jnp.maximum(ln * inv_sqrt_n, 0.0) + node


def _gin_post(node, aggs, p, n_norm, rows):
    body = functools.partial(_gin_post_body,
                             inv_sqrt_n=float(1.0 / np.sqrt(n_norm)),
                             n_agg=len(aggs))
    return _pc(
        body, (rows // _RB,),
        [_row_spec(32)] + [_row_spec(32)] * len(aggs) +
        [_full_spec((32, 64)), _full_spec((1, 64)),
         _full_spec((64, 32)), _full_spec((1, 32)),
         _full_spec((1, 32)), _full_spec((1, 32))],
        _row_spec(32),
        jax.ShapeDtypeStruct((rows, 32), jnp.float32),
    )(node, *aggs,
      p["mlp1"]["w"], p["mlp1"]["b"].reshape(1, 64),
      p["mlp2"]["w"], p["mlp2"]["b"].reshape(1, 32),
      p["gamma"].reshape(1, 32), p["beta"].reshape(1, 32))


def _mean_body(x_ref, o_ref):
    @pl.when(pl.program_id(0) == 0)
    def _():
        o_ref[...] = jnp.zeros_like(o_ref)
    o_ref[...] += jnp.sum(x_ref[...], axis=0, keepdims=True)


def _mean_pool(x, rows):
    s = _pc(
        _mean_body, (rows // _RB,),
        [_row_spec(32)], pl.BlockSpec((1, 32), lambda i: (0, 0)),
        jax.ShapeDtypeStruct((1, 32), jnp.float32),
    )(x)
    return s / float(rows)


_CH = 1024
_CR = 8
_EPAD = 800768
_NCH = 782
_BKR = 25000
_ACC = 25088
_ZR = 784
_SENT = 1 << 29


def _window_list(nbk_total, margin):
    win = []
    for k in range(nbk_total):
        s = (k * 800000) // nbk_total
        e = ((k + 1) * 800000) // nbk_total
        w0 = max(0, s // _CH - margin)
        w1 = min(_NCH, (e + _CH - 1) // _CH + margin)
        win.append((w0, w1))
    return win


_WIN_AB = _window_list(2, 12)
_WIN_BA = _window_list(16, 6)


def _make_agg_kernel(nbk, second_gather, out_rows, win):
    mesh = plsc.VectorSubcoreMesh(core_axis_name="c", subcore_axis_name="s")
    scratch = [
        pltpu.VMEM((_CR, 128), jnp.int32),
        pltpu.VMEM((_CR, 128), jnp.int32),
        pltpu.VMEM((_CR, 128), jnp.int32),
        pltpu.VMEM((_CR, 128), jnp.int32),
        pltpu.VMEM((_CH, 32), jnp.float32),
        pltpu.VMEM((_CH, 32), jnp.float32),
        pltpu.VMEM_SHARED((_ACC, 32), jnp.float32),
        pltpu.SemaphoreType.DMA,
    ]

    @functools.partial(
        pl.kernel, mesh=mesh,
        compiler_params=pltpu.CompilerParams(use_tc_tiling_on_sc=False),
        out_type=jax.ShapeDtypeStruct((out_rows, 32), jnp.float32),
        scratch_types=scratch,
    )
    def k(node_hbm, edge_hbm, srcb, eidb, dstb, zeros_hbm,
          out_hbm, srcv, eidv, dstv, idxv, gath, sec, acc, sem):
        cid = lax.axis_index("c")
        sid = lax.axis_index("s")

        for kk in range(nbk):
            bk = cid * nbk + kk
            base = bk * _BKR
            w0a, w1a = win[kk]
            w0b, w1b = win[nbk + kk]
            cs = w0a + cid * (w0b - w0a)
            ce = w1a + cid * (w1b - w1a)

            pltpu.sync_copy(zeros_hbm, gath.at[pl.ds(0, _ZR)])
            for p in range(2):
                pltpu.sync_copy(gath.at[pl.ds(0, _ZR)],
                                acc.at[pl.ds((sid * 2 + p) * _ZR, _ZR)])
            plsc.subcore_barrier()

            def chunk_body(i, _):
                c = cs + sid + i * 16
                pltpu.sync_copy(srcb.at[c], srcv)
                pltpu.sync_copy(dstb.at[c], dstv)
                hs = [pltpu.async_copy(node_hbm.at[srcv.at[j]],
                                       gath.at[pl.ds(j * 128, 128)], sem)
                      for j in range(_CR)]
                if second_gather:
                    pltpu.sync_copy(eidb.at[c], eidv)
                    hs += [pltpu.async_copy(edge_hbm.at[eidv.at[j]],
                                            sec.at[pl.ds(j * 128, 128)], sem)
                           for j in range(_CR)]
                else:
                    pltpu.sync_copy(edge_hbm.at[pl.ds(c * _CH, _CH)], sec)
                for h in hs:
                    h.wait()

                def row(r, _):
                    s0 = sec[r, pl.ds(0, 16)]
                    s1 = sec[r, pl.ds(16, 16)]
                    g0 = gath[r, pl.ds(0, 16)]
                    gath[r, pl.ds(0, 16)] = jnp.maximum(g0 + s0, 0.0)
                    g1 = gath[r, pl.ds(16, 16)]
                    gath[r, pl.ds(16, 16)] = jnp.maximum(g1 + s1, 0.0)
                    return 0
                lax.fori_loop(0, _CH, row, 0)

                for j in range(_CR):
                    for l in range(8):
                        d = dstv[j, pl.ds(l * 16, 16)]
                        ok = (d >= base) & (d < base + _BKR)
                        idxv[j, pl.ds(l * 16, 16)] = jnp.where(
                            ok, d - base, _BKR)
                    pltpu.sync_copy(gath.at[pl.ds(j * 128, 128)],
                                    acc.at[idxv.at[j]], add=True)
                return 0

            ntrip = (ce - cs - sid + 15) // 16
            lax.fori_loop(0, ntrip, chunk_body, 0)
            plsc.subcore_barrier()

            for pp in range(3):
                p = sid + 16 * pp

                @pl.when(p < 40)
                def _():
                    pltpu.sync_copy(acc.at[pl.ds(p * 625, 625)],
                                    gath.at[pl.ds(0, 625)])
                    pltpu.sync_copy(gath.at[pl.ds(0, 625)],
                                    out_hbm.at[pl.ds(base + p * 625, 625)])
            plsc.subcore_barrier()

    return k


def _concat_bond_tables(tables, rbf_lin):
    g = jnp.concatenate([jnp.concatenate(list(tables), axis=0),
                         rbf_lin["w"],
                         jnp.zeros((4, 32), jnp.float32)], axis=0)
    return g, rbf_lin["b"].reshape(1, 32)


def kernel(atom_feats, bond_feats, bond_float_feats, bond_angle_float_feats,
           ab_edge_index, ba_edge_index, params):
    n_atoms = atom_feats.shape[0]
    e_ab = bond_feats.shape[0]
    e_ba = bond_angle_float_feats.shape[0]
    n_bonds = e_ab // 2

    def _prep(dst, src, win):
        perm = jnp.argsort(dst)
        d_s = dst[perm].astype(jnp.int32)
        s_s = src[perm].astype(jnp.int32)
        dpad = jnp.full((_EPAD,), _SENT, jnp.int32).at[:d_s.shape[0]].set(d_s)
        spad = jnp.zeros((_EPAD,), jnp.int32).at[:s_s.shape[0]].set(s_s)
        bid = d_s // _BKR
        w0 = jnp.asarray([w[0] for w in win], jnp.int32)[bid]
        w1 = jnp.asarray([w[1] for w in win], jnp.int32)[bid]
        chunk = jnp.arange(d_s.shape[0], dtype=jnp.int32) // _CH
        covered = (chunk >= w0) & (chunk < w1)
        return (perm, spad.reshape(_NCH, _CR, 128),
                dpad.reshape(_NCH, _CR, 128), d_s, covered)

    def _pad_idx(v):
        return (jnp.zeros((_EPAD,), jnp.int32).at[:v.shape[0]]
                .set(v.astype(jnp.int32)).reshape(_NCH, _CR, 128))

    zeros_z = jnp.zeros((_ZR, 32), jnp.float32)
    perm_ab, srcab, dstab, dsab_s, cov_ab = _prep(
        ab_edge_index[1], ab_edge_index[0], _WIN_AB)
    eid0 = _pad_idx(perm_ab)
    eid1 = _pad_idx(perm_ab // 2)
    srab_s = ab_edge_index[0][perm_ab].astype(jnp.int32)
    perm_ba, srcba, dstba, dsba_s, cov_ba = _prep(
        ba_edge_index[1], ba_edge_index[0], _WIN_BA)
    srba_s = ba_edge_index[0][perm_ba].astype(jnp.int32)
    x_s = bond_angle_float_feats[perm_ba]
    miss_ab = jnp.any(~cov_ab)
    miss_ba = jnp.any(~cov_ba)

    def _fallback(agg, miss, cov, node_tab, edge_term_fn, src_s, dst_s, rows):
        def fb(a):
            msg = jnp.maximum(node_tab[src_s] + edge_term_fn(), 0.0)
            msg = jnp.where(cov[:, None], 0.0, msg)
            return a + jax.ops.segment_sum(msg, dst_s, num_segments=rows)
        return lax.cond(miss, fb, lambda a: a, agg)

    ab_k0 = _make_agg_kernel(1, True, n_atoms, _WIN_AB)
    ab_k = _make_agg_kernel(1, True, n_atoms, _WIN_AB)
    ba_k = _make_agg_kernel(8, False, n_bonds, _WIN_BA)

    atom_w = jnp.concatenate(list(params["init_atom_emb"]), axis=0)
    node_repr = _atom_embed(atom_feats, atom_w, n_atoms)

    feats = _featurize(bond_feats, bond_float_feats, e_ab)
    feats_even = feats[::2]

    g0, b0 = _concat_bond_tables(params["init_bond_emb"],
                                 params["init_bond_rbf"])
    edge_full = _mm(feats, g0, b0, e_ab, 48)

    edge_half = None
    for lp in params["layers"]:
        if edge_half is None:
            agg = ab_k0(node_repr, edge_full, srcab, eid0, dstab, zeros_z)
            et_fn = functools.partial(lambda e: e[perm_ab], edge_full)
        else:
            agg = ab_k(node_repr, edge_half, srcab, eid1, dstab, zeros_z)
            et_fn = functools.partial(lambda e: e[perm_ab // 2], edge_half)
        agg = _fallback(agg, miss_ab, cov_ab, node_repr, et_fn,
                        srab_s, dsab_s, n_atoms)
        node_out = _gin_post(node_repr, [agg], lp["ab_gnn"], n_atoms, n_atoms)

        gl, bl = _concat_bond_tables(lp["bond_emb"], lp["bond_rbf"])
        bond_embed = _mm(feats_even, gl, bl, n_bonds, 48)
        ba_emb = _ba_embed(x_s, lp["ba_rbf"]["w"],
                           lp["ba_rbf"]["b"].reshape(1, 32), e_ba)
        agg_b = ba_k(bond_embed, ba_emb, srcba, srcba, dstba, zeros_z)
        agg_b = _fallback(agg_b, miss_ba, cov_ba, bond_embed, lambda: ba_emb,
                          srba_s, dsba_s, n_bonds)
        edge_half = _gin_post(bond_embed, [agg_b], lp["ba_gnn"],
                              n_bonds, n_bonds)

        node_repr = node_out

    edge_repr = jnp.repeat(edge_half, 2, axis=0)
    graph_repr = _mean_pool(node_repr, n_atoms)
    return node_repr, edge_repr, graph_repr


# speedup vs baseline: 17.2863x; 1.0314x over previous
import functools
import jax
import jax.numpy as jnp
import numpy as np
from jax import lax
from jax.experimental import pallas as pl
from jax.experimental.pallas import tpu as pltpu
from jax.experimental.pallas import tpu_sc as plsc

_RB = 2000
_INTERPRET = False


def _pc(body, grid, in_specs, out_specs, out_shape):
    return pl.pallas_call(
        body, grid=grid, in_specs=in_specs, out_specs=out_specs,
        out_shape=out_shape, interpret=_INTERPRET)


def _row_spec(cols):
    return pl.BlockSpec((_RB, cols), lambda i: (i, 0))


def _full_spec(shape):
    return pl.BlockSpec(shape, lambda i: (0, 0))


def _featurize_body(bf_ref, bx_ref, o_ref):
    pieces = []
    for i in range(3):
        f = bf_ref[:, i:i + 1]
        oh = (f == jax.lax.broadcasted_iota(jnp.int32, (1, 8), 1))
        pieces.append(oh.astype(jnp.float32))
    x = bx_ref[:, 0:1]
    cent = (jax.lax.broadcasted_iota(jnp.int32, (1, 20), 1)
            .astype(jnp.float32) * (2.0 / 19.0))
    e = jnp.exp(-10.0 * (x - cent) ** 2)
    z = jnp.zeros((bf_ref.shape[0], 4), jnp.float32)
    o_ref[...] = jnp.concatenate(pieces + [e, z], axis=1)


def _featurize(bond_feats, bond_float, rows):
    return _pc(
        _featurize_body, (rows // _RB,),
        [_row_spec(3), _row_spec(1)], _row_spec(48),
        jax.ShapeDtypeStruct((rows, 48), jnp.float32),
    )(bond_feats, bond_float)


def _atom_body(af_ref, w_ref, o_ref):
    pieces = []
    for i in range(9):
        f = af_ref[:, i:i + 1]
        oh = (f == jax.lax.broadcasted_iota(jnp.int32, (1, 16), 1))
        pieces.append(oh.astype(jnp.float32))
    x = jnp.concatenate(pieces, axis=1)
    o_ref[...] = jnp.dot(x, w_ref[...], preferred_element_type=jnp.float32)


def _atom_embed(atom_feats, w, rows):
    return _pc(
        _atom_body, (rows // _RB,),
        [_row_spec(9), _full_spec((144, 32))], _row_spec(32),
        jax.ShapeDtypeStruct((rows, 32), jnp.float32),
    )(atom_feats, w)


def _mm_body(x_ref, w_ref, b_ref, o_ref):
    o_ref[...] = (jnp.dot(x_ref[...], w_ref[...],
                          preferred_element_type=jnp.float32) + b_ref[...])


def _mm(x, w, b, rows, k):
    return _pc(
        _mm_body, (rows // _RB,),
        [_row_spec(k), _full_spec((k, 32)), _full_spec((1, 32))],
        _row_spec(32),
        jax.ShapeDtypeStruct((rows, 32), jnp.float32),
    )(x, w, b)


def _ba_embed_body(x_ref, w_ref, b_ref, o_ref):
    x = x_ref[:, 0:1]
    cent = (jax.lax.broadcasted_iota(jnp.int32, (1, 20), 1)
            .astype(jnp.float32) * (2.0 / 19.0))
    e = jnp.exp(-10.0 * (x - cent) ** 2)
    o_ref[...] = (jnp.dot(e, w_ref[...],
                          preferred_element_type=jnp.float32) + b_ref[...])


def _ba_embed(x, w, b, rows):
    return _pc(
        _ba_embed_body, (rows // _RB,),
        [_row_spec(1), _full_spec((20, 32)), _full_spec((1, 32))],
        _row_spec(32),
        jax.ShapeDtypeStruct((rows, 32), jnp.float32),
    )(x, w, b)


def _gin_post_body(n_ref, *refs, inv_sqrt_n, n_agg):
    agg_refs = refs[:n_agg]
    (w1_ref, b1_ref, w2_ref, b2_ref, g_ref, bt_ref, o_ref) = refs[n_agg:]
    node = n_ref[...]
    h = node
    for a_ref in agg_refs:
        h = h + a_ref[...]
    a = jnp.maximum(jnp.dot(h, w1_ref[...],
                            preferred_element_type=jnp.float32) + b1_ref[...],
                    0.0)
    z = (jnp.dot(a, w2_ref[...], preferred_element_type=jnp.float32)
         + b2_ref[...])
    m = jnp.mean(z, axis=-1, keepdims=True)
    v = jnp.mean((z - m) ** 2, axis=-1, keepdims=True)
    ln = (z - m) / jnp.sqrt(v + 1e-5) * g_ref[...] + bt_ref[...]
    o_ref[...] = jnp.maximum(ln * inv_sqrt_n, 0.0) + node


def _gin_post(node, aggs, p, n_norm, rows):
    body = functools.partial(_gin_post_body,
                             inv_sqrt_n=float(1.0 / np.sqrt(n_norm)),
                             n_agg=len(aggs))
    return _pc(
        body, (rows // _RB,),
        [_row_spec(32)] + [_row_spec(32)] * len(aggs) +
        [_full_spec((32, 64)), _full_spec((1, 64)),
         _full_spec((64, 32)), _full_spec((1, 32)),
         _full_spec((1, 32)), _full_spec((1, 32))],
        _row_spec(32),
        jax.ShapeDtypeStruct((rows, 32), jnp.float32),
    )(node, *aggs,
      p["mlp1"]["w"], p["mlp1"]["b"].reshape(1, 64),
      p["mlp2"]["w"], p["mlp2"]["b"].reshape(1, 32),
      p["gamma"].reshape(1, 32), p["beta"].reshape(1, 32))


def _mean_body(x_ref, o_ref):
    @pl.when(pl.program_id(0) == 0)
    def _():
        o_ref[...] = jnp.zeros_like(o_ref)
    o_ref[...] += jnp.sum(x_ref[...], axis=0, keepdims=True)


def _mean_pool(x, rows):
    s = _pc(
        _mean_body, (rows // _RB,),
        [_row_spec(32)], pl.BlockSpec((1, 32), lambda i: (0, 0)),
        jax.ShapeDtypeStruct((1, 32), jnp.float32),
    )(x)
    return s / float(rows)


_CH = 1024
_CR = 8
_EPAD = 800768
_NCH = 782
_BKR = 25000
_ACC = 25088
_ZR = 784
_SENT = 1 << 29


def _window_list(nbk_total, margin):
    win = []
    for k in range(nbk_total):
        s = (k * 800000) // nbk_total
        e = ((k + 1) * 800000) // nbk_total
        w0 = max(0, s // _CH - margin)
        w1 = min(_NCH, (e + _CH - 1) // _CH + margin)
        win.append((w0, w1))
    return win


_WIN_AB = _window_list(2, 12)
_WIN_BA = _window_list(16, 6)


def _make_agg_kernel(nbk, second_gather, out_rows, win):
    mesh = plsc.VectorSubcoreMesh(core_axis_name="c", subcore_axis_name="s")
    scratch = [
        pltpu.VMEM((_CR, 128), jnp.int32),
        pltpu.VMEM((_CR, 128), jnp.int32),
        pltpu.VMEM((_CR, 128), jnp.int32),
        pltpu.VMEM((_CR, 128), jnp.int32),
        pltpu.VMEM((_CH, 32), jnp.float32),
        pltpu.VMEM((_CH, 32), jnp.float32),
        pltpu.VMEM_SHARED((_ACC, 32), jnp.float32),
        pltpu.SemaphoreType.DMA,
    ]

    @functools.partial(
        pl.kernel, mesh=mesh,
        compiler_params=pltpu.CompilerParams(use_tc_tiling_on_sc=False),
        out_type=jax.ShapeDtypeStruct((out_rows, 32), jnp.float32),
        scratch_types=scratch,
    )
    def k(node_hbm, edge_hbm, srcb, eidb, dstb, zeros_hbm,
          out_hbm, srcv, eidv, dstv, idxv, gath, sec, acc, sem):
        cid = lax.axis_index("c")
        sid = lax.axis_index("s")

        for kk in range(nbk):
            bk = cid * nbk + kk
            base = bk * _BKR
            w0a, w1a = win[kk]
            w0b, w1b = win[nbk + kk]
            cs = w0a + cid * (w0b - w0a)
            ce = w1a + cid * (w1b - w1a)

            pltpu.sync_copy(zeros_hbm, gath.at[pl.ds(0, _ZR)])
            for p in range(2):
                pltpu.sync_copy(gath.at[pl.ds(0, _ZR)],
                                acc.at[pl.ds((sid * 2 + p) * _ZR, _ZR)])
            plsc.subcore_barrier()

            def chunk_body(i, _):
                c = cs + sid + i * 16
                pltpu.sync_copy(srcb.at[c], srcv)
                pltpu.sync_copy(dstb.at[c], dstv)
                hs = [pltpu.async_copy(node_hbm.at[srcv.at[j]],
                                       gath.at[pl.ds(j * 128, 128)], sem)
                      for j in range(_CR)]
                if second_gather:
                    pltpu.sync_copy(eidb.at[c], eidv)
                    hs += [pltpu.async_copy(edge_hbm.at[eidv.at[j]],
                                            sec.at[pl.ds(j * 128, 128)], sem)
                           for j in range(_CR)]
                else:
                    pltpu.sync_copy(edge_hbm.at[pl.ds(c * _CH, _CH)], sec)
                for h in hs:
                    h.wait()

                def row(r4, _):
                    r = r4 * 8
                    for d in range(8):
                        rr = r + d
                        s0 = sec[rr, pl.ds(0, 16)]
                        s1 = sec[rr, pl.ds(16, 16)]
                        g0 = gath[rr, pl.ds(0, 16)]
                        gath[rr, pl.ds(0, 16)] = jnp.maximum(g0 + s0, 0.0)
                        g1 = gath[rr, pl.ds(16, 16)]
                        gath[rr, pl.ds(16, 16)] = jnp.maximum(g1 + s1, 0.0)
                    return 0
                lax.fori_loop(0, _CH // 8, row, 0)

                for j in range(_CR):
                    for l in range(8):
                        d = dstv[j, pl.ds(l * 16, 16)]
                        ok = (d >= base) & (d < base + _BKR)
                        idxv[j, pl.ds(l * 16, 16)] = jnp.where(
                            ok, d - base, _BKR)
                    pltpu.sync_copy(gath.at[pl.ds(j * 128, 128)],
                                    acc.at[idxv.at[j]], add=True)
                return 0

            ntrip = (ce - cs - sid + 15) // 16
            lax.fori_loop(0, ntrip, chunk_body, 0)
            plsc.subcore_barrier()

            for pp in range(3):
                p = sid + 16 * pp

                @pl.when(p < 40)
                def _():
                    pltpu.sync_copy(acc.at[pl.ds(p * 625, 625)],
                                    gath.at[pl.ds(0, 625)])
                    pltpu.sync_copy(gath.at[pl.ds(0, 625)],
                                    out_hbm.at[pl.ds(base + p * 625, 625)])
            plsc.subcore_barrier()

    return k


def _concat_bond_tables(tables, rbf_lin):
    g = jnp.concatenate([jnp.concatenate(list(tables), axis=0),
                         rbf_lin["w"],
                         jnp.zeros((4, 32), jnp.float32)], axis=0)
    return g, rbf_lin["b"].reshape(1, 32)


def kernel(atom_feats, bond_feats, bond_float_feats, bond_angle_float_feats,
           ab_edge_index, ba_edge_index, params):
    n_atoms = atom_feats.shape[0]
    e_ab = bond_feats.shape[0]
    e_ba = bond_angle_float_feats.shape[0]
    n_bonds = e_ab // 2

    def _prep(dst, src, win):
        perm = jnp.argsort(dst)
        d_s = dst[perm].astype(jnp.int32)
        s_s = src[perm].astype(jnp.int32)
        dpad = jnp.full((_EPAD,), _SENT, jnp.int32).at[:d_s.shape[0]].set(d_s)
        spad = jnp.zeros((_EPAD,), jnp.int32).at[:s_s.shape[0]].set(s_s)
        bid = d_s // _BKR
        w0 = jnp.asarray([w[0] for w in win], jnp.int32)[bid]
        w1 = jnp.asarray([w[1] for w in win], jnp.int32)[bid]
        chunk = jnp.arange(d_s.shape[0], dtype=jnp.int32) // _CH
        covered = (chunk >= w0) & (chunk < w1)
        return (perm, spad.reshape(_NCH, _CR, 128),
                dpad.reshape(_NCH, _CR, 128), d_s, covered)

    def _pad_idx(v):
        return (jnp.zeros((_EPAD,), jnp.int32).at[:v.shape[0]]
                .set(v.astype(jnp.int32)).reshape(_NCH, _CR, 128))

    zeros_z = jnp.zeros((_ZR, 32), jnp.float32)
    perm_ab, srcab, dstab, dsab_s, cov_ab = _prep(
        ab_edge_index[1], ab_edge_index[0], _WIN_AB)
    eid0 = _pad_idx(perm_ab)
    eid1 = _pad_idx(perm_ab // 2)
    srab_s = ab_edge_index[0][perm_ab].astype(jnp.int32)
    perm_ba, srcba, dstba, dsba_s, cov_ba = _prep(
        ba_edge_index[1], ba_edge_index[0], _WIN_BA)
    srba_s = ba_edge_index[0][perm_ba].astype(jnp.int32)
    x_s = bond_angle_float_feats[perm_ba]
    miss_ab = jnp.any(~cov_ab)
    miss_ba = jnp.any(~cov_ba)

    def _fallback(agg, miss, cov, node_tab, edge_term_fn, src_s, dst_s, rows):
        def fb(a):
            msg = jnp.maximum(node_tab[src_s] + edge_term_fn(), 0.0)
            msg = jnp.where(cov[:, None], 0.0, msg)
            return a + jax.ops.segment_sum(msg, dst_s, num_segments=rows)
        return lax.cond(miss, fb, lambda a: a, agg)

    ab_k0 = _make_agg_kernel(1, True, n_atoms, _WIN_AB)
    ab_k = _make_agg_kernel(1, True, n_atoms, _WIN_AB)
    ba_k = _make_agg_kernel(8, False, n_bonds, _WIN_BA)

    atom_w = jnp.concatenate(list(params["init_atom_emb"]), axis=0)
    node_repr = _atom_embed(atom_feats, atom_w, n_atoms)

    feats = _featurize(bond_feats, bond_float_feats, e_ab)
    feats_even = feats[::2]

    g0, b0 = _concat_bond_tables(params["init_bond_emb"],
                                 params["init_bond_rbf"])
    edge_full = _mm(feats, g0, b0, e_ab, 48)

    edge_half = None
    for lp in params["layers"]:
        if edge_half is None:
            agg = ab_k0(node_repr, edge_full, srcab, eid0, dstab, zeros_z)
            et_fn = functools.partial(lambda e: e[perm_ab], edge_full)
        else:
            agg = ab_k(node_repr, edge_half, srcab, eid1, dstab, zeros_z)
            et_fn = functools.partial(lambda e: e[perm_ab // 2], edge_half)
        agg = _fallback(agg, miss_ab, cov_ab, node_repr, et_fn,
                        srab_s, dsab_s, n_atoms)
        node_out = _gin_post(node_repr, [agg], lp["ab_gnn"], n_atoms, n_atoms)

        gl, bl = _concat_bond_tables(lp["bond_emb"], lp["bond_rbf"])
        bond_embed = _mm(feats_even, gl, bl, n_bonds, 48)
        ba_emb = _ba_embed(x_s, lp["ba_rbf"]["w"],
                           lp["ba_rbf"]["b"].reshape(1, 32), e_ba)
        agg_b = ba_k(bond_embed, ba_emb, srcba, srcba, dstba, zeros_z)
        agg_b = _fallback(agg_b, miss_ba, cov_ba, bond_embed, lambda: ba_emb,
                          srba_s, dsba_s, n_bonds)
        edge_half = _gin_post(bond_embed, [agg_b], lp["ba_gnn"],
                              n_bonds, n_bonds)

        node_repr = node_out

    edge_repr = jnp.repeat(edge_half, 2, axis=0)
    graph_repr = _mean_pool(node_repr, n_atoms)
    return node_repr, edge_repr, graph_repr


# speedup vs baseline: 17.4844x; 1.0115x over previous
import functools
import jax
import jax.numpy as jnp
import numpy as np
from jax import lax
from jax.experimental import pallas as pl
from jax.experimental.pallas import tpu as pltpu
from jax.experimental.pallas import tpu_sc as plsc

_RB = 2000
_INTERPRET = False


def _pc(body, grid, in_specs, out_specs, out_shape):
    return pl.pallas_call(
        body, grid=grid, in_specs=in_specs, out_specs=out_specs,
        out_shape=out_shape, interpret=_INTERPRET)


def _row_spec(cols):
    return pl.BlockSpec((_RB, cols), lambda i: (i, 0))


def _full_spec(shape):
    return pl.BlockSpec(shape, lambda i: (0, 0))


def _featurize_body(bf_ref, bx_ref, o_ref):
    pieces = []
    for i in range(3):
        f = bf_ref[:, i:i + 1]
        oh = (f == jax.lax.broadcasted_iota(jnp.int32, (1, 8), 1))
        pieces.append(oh.astype(jnp.float32))
    x = bx_ref[:, 0:1]
    cent = (jax.lax.broadcasted_iota(jnp.int32, (1, 20), 1)
            .astype(jnp.float32) * (2.0 / 19.0))
    e = jnp.exp(-10.0 * (x - cent) ** 2)
    z = jnp.zeros((bf_ref.shape[0], 4), jnp.float32)
    o_ref[...] = jnp.concatenate(pieces + [e, z], axis=1)


def _featurize(bond_feats, bond_float, rows):
    return _pc(
        _featurize_body, (rows // _RB,),
        [_row_spec(3), _row_spec(1)], _row_spec(48),
        jax.ShapeDtypeStruct((rows, 48), jnp.float32),
    )(bond_feats, bond_float)


def _atom_body(af_ref, w_ref, o_ref):
    pieces = []
    for i in range(9):
        f = af_ref[:, i:i + 1]
        oh = (f == jax.lax.broadcasted_iota(jnp.int32, (1, 16), 1))
        pieces.append(oh.astype(jnp.float32))
    x = jnp.concatenate(pieces, axis=1)
    o_ref[...] = jnp.dot(x, w_ref[...], preferred_element_type=jnp.float32)


def _atom_embed(atom_feats, w, rows):
    return _pc(
        _atom_body, (rows // _RB,),
        [_row_spec(9), _full_spec((144, 32))], _row_spec(32),
        jax.ShapeDtypeStruct((rows, 32), jnp.float32),
    )(atom_feats, w)


def _mm_body(x_ref, w_ref, b_ref, o_ref):
    o_ref[...] = (jnp.dot(x_ref[...], w_ref[...],
                          preferred_element_type=jnp.float32) + b_ref[...])


def _mm(x, w, b, rows, k):
    return _pc(
        _mm_body, (rows // _RB,),
        [_row_spec(k), _full_spec((k, 32)), _full_spec((1, 32))],
        _row_spec(32),
        jax.ShapeDtypeStruct((rows, 32), jnp.float32),
    )(x, w, b)


def _ba_embed_body(x_ref, w_ref, b_ref, o_ref):
    x = x_ref[:, 0:1]
    cent = (jax.lax.broadcasted_iota(jnp.int32, (1, 20), 1)
            .astype(jnp.float32) * (2.0 / 19.0))
    e = jnp.exp(-10.0 * (x - cent) ** 2)
    o_ref[...] = (jnp.dot(e, w_ref[...],
                          preferred_element_type=jnp.float32) + b_ref[...])


def _ba_embed(x, w, b, rows):
    return _pc(
        _ba_embed_body, (rows // _RB,),
        [_row_spec(1), _full_spec((20, 32)), _full_spec((1, 32))],
        _row_spec(32),
        jax.ShapeDtypeStruct((rows, 32), jnp.float32),
    )(x, w, b)


def _gin_post_body(n_ref, *refs, inv_sqrt_n, n_agg):
    agg_refs = refs[:n_agg]
    (w1_ref, b1_ref, w2_ref, b2_ref, g_ref, bt_ref, o_ref) = refs[n_agg:]
    node = n_ref[...]
    h = node
    for a_ref in agg_refs:
        h = h + a_ref[...]
    a = jnp.maximum(jnp.dot(h, w1_ref[...],
                            preferred_element_type=jnp.float32) + b1_ref[...],
                    0.0)
    z = (jnp.dot(a, w2_ref[...], preferred_element_type=jnp.float32)
         + b2_ref[...])
    m = jnp.mean(z, axis=-1, keepdims=True)
    v = jnp.mean((z - m) ** 2, axis=-1, keepdims=True)
    ln = (z - m) / jnp.sqrt(v + 1e-5) * g_ref[...] + bt_ref[...]
    o_ref[...] = jnp.maximum(ln * inv_sqrt_n, 0.0) + node


def _gin_post(node, aggs, p, n_norm, rows):
    body = functools.partial(_gin_post_body,
                             inv_sqrt_n=float(1.0 / np.sqrt(n_norm)),
                             n_agg=len(aggs))
    return _pc(
        body, (rows // _RB,),
        [_row_spec(32)] + [_row_spec(32)] * len(aggs) +
        [_full_spec((32, 64)), _full_spec((1, 64)),
         _full_spec((64, 32)), _full_spec((1, 32)),
         _full_spec((1, 32)), _full_spec((1, 32))],
        _row_spec(32),
        jax.ShapeDtypeStruct((rows, 32), jnp.float32),
    )(node, *aggs,
      p["mlp1"]["w"], p["mlp1"]["b"].reshape(1, 64),
      p["mlp2"]["w"], p["mlp2"]["b"].reshape(1, 32),
      p["gamma"].reshape(1, 32), p["beta"].reshape(1, 32))


def _mean_body(x_ref, o_ref):
    @pl.when(pl.program_id(0) == 0)
    def _():
        o_ref[...] = jnp.zeros_like(o_ref)
    o_ref[...] += jnp.sum(x_ref[...], axis=0, keepdims=True)


def _mean_pool(x, rows):
    s = _pc(
        _mean_body, (rows // _RB,),
        [_row_spec(32)], pl.BlockSpec((1, 32), lambda i: (0, 0)),
        jax.ShapeDtypeStruct((1, 32), jnp.float32),
    )(x)
    return s / float(rows)


_CH = 1024
_CR = 8
_EPAD = 800768
_NCH = 782
_BKR = 25000
_ACC = 25088
_ZR = 784
_SENT = 1 << 29


def _window_list(nbk_total, margin):
    win = []
    for k in range(nbk_total):
        s = (k * 800000) // nbk_total
        e = ((k + 1) * 800000) // nbk_total
        w0 = max(0, s // _CH - margin)
        w1 = min(_NCH, (e + _CH - 1) // _CH + margin)
        win.append((w0, w1))
    return win


_WIN_AB = _window_list(2, 12)
_WIN_BA = _window_list(16, 6)


def _make_agg_kernel(nbk, second_gather, out_rows, win):
    mesh = plsc.VectorSubcoreMesh(core_axis_name="c", subcore_axis_name="s")
    nir = 3 * _CR if second_gather else 2 * _CR
    scratch = [
        pltpu.VMEM((nir, 128), jnp.int32),
        pltpu.VMEM((_CR, 128), jnp.int32),
        pltpu.VMEM((_CH, 32), jnp.float32),
        pltpu.VMEM((_CH, 32), jnp.float32),
        pltpu.VMEM_SHARED((_ACC, 32), jnp.float32),
        pltpu.SemaphoreType.DMA,
        pltpu.SemaphoreType.DMA,
    ]

    @functools.partial(
        pl.kernel, mesh=mesh,
        compiler_params=pltpu.CompilerParams(use_tc_tiling_on_sc=False),
        out_type=jax.ShapeDtypeStruct((out_rows, 32), jnp.float32),
        scratch_types=scratch,
    )
    def k(node_hbm, edge_hbm, idxb, zeros_hbm,
          out_hbm, pidv, idxv, gath, sec, acc, sem, sem2):
        cid = lax.axis_index("c")
        sid = lax.axis_index("s")

        for kk in range(nbk):
            bk = cid * nbk + kk
            base = bk * _BKR
            w0a, w1a = win[kk]
            w0b, w1b = win[nbk + kk]
            cs = w0a + cid * (w0b - w0a)
            ce = w1a + cid * (w1b - w1a)

            pltpu.sync_copy(zeros_hbm, gath.at[pl.ds(0, _ZR)])
            for p in range(2):
                pltpu.sync_copy(gath.at[pl.ds(0, _ZR)],
                                acc.at[pl.ds((sid * 2 + p) * _ZR, _ZR)])
            plsc.subcore_barrier()

            def chunk_body(i, _):
                c = cs + sid + i * 16
                pltpu.sync_copy(idxb.at[c], pidv)
                hs = [pltpu.async_copy(node_hbm.at[pidv.at[j]],
                                       gath.at[pl.ds(j * 128, 128)], sem)
                      for j in range(_CR)]
                if second_gather:
                    hs += [pltpu.async_copy(edge_hbm.at[pidv.at[_CR + j]],
                                            sec.at[pl.ds(j * 128, 128)], sem)
                           for j in range(_CR)]
                else:
                    hs.append(pltpu.async_copy(
                        edge_hbm.at[pl.ds(c * _CH, _CH)], sec, sem))
                for h in hs:
                    h.wait()

                def row(r4, _):
                    r = r4 * 8
                    for d in range(8):
                        rr = r + d
                        s0 = sec[rr, pl.ds(0, 16)]
                        s1 = sec[rr, pl.ds(16, 16)]
                        g0 = gath[rr, pl.ds(0, 16)]
                        gath[rr, pl.ds(0, 16)] = jnp.maximum(g0 + s0, 0.0)
                        g1 = gath[rr, pl.ds(16, 16)]
                        gath[rr, pl.ds(16, 16)] = jnp.maximum(g1 + s1, 0.0)
                    return 0
                lax.fori_loop(0, _CH // 8, row, 0)

                doff = 2 * _CR if second_gather else _CR
                for j in range(_CR):
                    for l in range(8):
                        d = pidv[doff + j, pl.ds(l * 16, 16)]
                        ok = (d >= base) & (d < base + _BKR)
                        idxv[j, pl.ds(l * 16, 16)] = jnp.where(
                            ok, d - base, _BKR)
                ws = [pltpu.async_copy(gath.at[pl.ds(j * 128, 128)],
                                       acc.at[idxv.at[j]], sem2, add=True)
                      for j in range(_CR)]
                for h in ws:
                    h.wait()
                return 0

            ntrip = (ce - cs - sid + 15) // 16
            lax.fori_loop(0, ntrip, chunk_body, 0)
            plsc.subcore_barrier()

            for pp in range(3):
                p = sid + 16 * pp

                @pl.when(p < 40)
                def _():
                    pltpu.sync_copy(acc.at[pl.ds(p * 625, 625)],
                                    gath.at[pl.ds(0, 625)])
                    pltpu.sync_copy(gath.at[pl.ds(0, 625)],
                                    out_hbm.at[pl.ds(base + p * 625, 625)])
            plsc.subcore_barrier()

    return k


def _concat_bond_tables(tables, rbf_lin):
    g = jnp.concatenate([jnp.concatenate(list(tables), axis=0),
                         rbf_lin["w"],
                         jnp.zeros((4, 32), jnp.float32)], axis=0)
    return g, rbf_lin["b"].reshape(1, 32)


def kernel(atom_feats, bond_feats, bond_float_feats, bond_angle_float_feats,
           ab_edge_index, ba_edge_index, params):
    n_atoms = atom_feats.shape[0]
    e_ab = bond_feats.shape[0]
    e_ba = bond_angle_float_feats.shape[0]
    n_bonds = e_ab // 2

    def _prep(dst, src, win):
        perm = jnp.argsort(dst)
        d_s = dst[perm].astype(jnp.int32)
        s_s = src[perm].astype(jnp.int32)
        dpad = jnp.full((_EPAD,), _SENT, jnp.int32).at[:d_s.shape[0]].set(d_s)
        spad = jnp.zeros((_EPAD,), jnp.int32).at[:s_s.shape[0]].set(s_s)
        bid = d_s // _BKR
        w0 = jnp.asarray([w[0] for w in win], jnp.int32)[bid]
        w1 = jnp.asarray([w[1] for w in win], jnp.int32)[bid]
        chunk = jnp.arange(d_s.shape[0], dtype=jnp.int32) // _CH
        covered = (chunk >= w0) & (chunk < w1)
        return (perm, spad.reshape(_NCH, _CR, 128),
                dpad.reshape(_NCH, _CR, 128), d_s, covered)

    def _pad_idx(v):
        return (jnp.zeros((_EPAD,), jnp.int32).at[:v.shape[0]]
                .set(v.astype(jnp.int32)).reshape(_NCH, _CR, 128))

    zeros_z = jnp.zeros((_ZR, 32), jnp.float32)
    perm_ab, srcab, dstab, dsab_s, cov_ab = _prep(
        ab_edge_index[1], ab_edge_index[0], _WIN_AB)
    eid0 = _pad_idx(perm_ab)
    eid1 = _pad_idx(perm_ab // 2)
    pack_ab0 = jnp.concatenate([srcab, eid0, dstab], axis=1)
    pack_ab1 = jnp.concatenate([srcab, eid1, dstab], axis=1)
    srab_s = ab_edge_index[0][perm_ab].astype(jnp.int32)
    perm_ba, srcba, dstba, dsba_s, cov_ba = _prep(
        ba_edge_index[1], ba_edge_index[0], _WIN_BA)
    pack_ba = jnp.concatenate([srcba, dstba], axis=1)
    srba_s = ba_edge_index[0][perm_ba].astype(jnp.int32)
    x_s = bond_angle_float_feats[perm_ba]
    miss_ab = jnp.any(~cov_ab)
    miss_ba = jnp.any(~cov_ba)

    def _fallback(agg, miss, cov, node_tab, edge_term_fn, src_s, dst_s, rows):
        def fb(a):
            msg = jnp.maximum(node_tab[src_s] + edge_term_fn(), 0.0)
            msg = jnp.where(cov[:, None], 0.0, msg)
            return a + jax.ops.segment_sum(msg, dst_s, num_segments=rows)
        return lax.cond(miss, fb, lambda a: a, agg)

    ab_k0 = _make_agg_kernel(1, True, n_atoms, _WIN_AB)
    ab_k = _make_agg_kernel(1, True, n_atoms, _WIN_AB)
    ba_k = _make_agg_kernel(8, False, n_bonds, _WIN_BA)

    atom_w = jnp.concatenate(list(params["init_atom_emb"]), axis=0)
    node_repr = _atom_embed(atom_feats, atom_w, n_atoms)

    feats = _featurize(bond_feats, bond_float_feats, e_ab)
    feats_even = feats[::2]

    g0, b0 = _concat_bond_tables(params["init_bond_emb"],
                                 params["init_bond_rbf"])
    edge_full = _mm(feats, g0, b0, e_ab, 48)

    edge_half = None
    for lp in params["layers"]:
        gl, bl = _concat_bond_tables(lp["bond_emb"], lp["bond_rbf"])
        bond_embed = _mm(feats_even, gl, bl, n_bonds, 48)
        ba_emb = _ba_embed(x_s, lp["ba_rbf"]["w"],
                           lp["ba_rbf"]["b"].reshape(1, 32), e_ba)

        if edge_half is None:
            agg = ab_k0(node_repr, edge_full, pack_ab0, zeros_z)
            et_fn = functools.partial(lambda e: e[perm_ab], edge_full)
        else:
            agg = ab_k(node_repr, edge_half, pack_ab1, zeros_z)
            et_fn = functools.partial(lambda e: e[perm_ab // 2], edge_half)
        agg = _fallback(agg, miss_ab, cov_ab, node_repr, et_fn,
                        srab_s, dsab_s, n_atoms)
        node_out = _gin_post(node_repr, [agg], lp["ab_gnn"], n_atoms, n_atoms)

        agg_b = ba_k(bond_embed, ba_emb, pack_ba, zeros_z)
        agg_b = _fallback(agg_b, miss_ba, cov_ba, bond_embed, lambda: ba_emb,
                          srba_s, dsba_s, n_bonds)
        edge_half = _gin_post(bond_embed, [agg_b], lp["ba_gnn"],
                              n_bonds, n_bonds)

        node_repr = node_out

    edge_repr = jnp.repeat(edge_half, 2, axis=0)
    graph_repr = _mean_pool(node_repr, n_atoms)
    return node_repr, edge_repr, graph_repr
